# Initial kernel scaffold; baseline (speedup 1.0000x reference)
import jax
import jax.numpy as jnp
from jax.experimental import pallas as pl


def kernel(token_ids, W, b):
    raise NotImplementedError("write your pallas kernel here")


# speedup vs baseline: 23.1321x; 23.1321x over previous
import functools

import jax
import jax.numpy as jnp
from jax import lax
from jax.experimental import pallas as pl
from jax.experimental.pallas import tpu as pltpu
from jax.experimental.pallas import tpu_sc as plsc

BATCH = 4096
SEQ = 200
VOCAB = 512
DMODEL = 1024

_NC = 2
_NS = 16
_NW = _NC * _NS
_L = 16

_ROWS_PER_W = BATCH // _NW
_GROUPS = _ROWS_PER_W // _L


def _hist_body(tok_hbm, counts_hbm, tok_v, hist_v, sem):
    wid = lax.axis_index("s") * _NC + lax.axis_index("c")
    lane = lax.iota(jnp.int32, _L)
    tok_base = lane * SEQ
    row_off = lane * VOCAB
    ones = jnp.ones((_L,), jnp.float32)
    zeros = jnp.zeros((_L,), jnp.float32)

    for g in range(_GROUPS):
        base = (wid * _GROUPS + g) * _L

        pltpu.sync_copy(tok_hbm.at[pl.ds(base * SEQ, _L * SEQ)], tok_v)

        def _zero(i, _):
            hist_v[pl.ds(i * _L, _L)] = zeros
            return 0
        lax.fori_loop(0, (_L * VOCAB) // _L, _zero, 0, unroll=8)

        def _step(s, _):
            tok = plsc.load_gather(tok_v, [tok_base + s])
            plsc.addupdate_scatter(hist_v, [row_off + tok], ones)
            return 0
        lax.fori_loop(0, SEQ, _step, 0, unroll=8)

        pltpu.sync_copy(hist_v, counts_hbm.at[pl.ds(base * VOCAB, _L * VOCAB)])


_hist = functools.partial(
    pl.kernel,
    mesh=plsc.VectorSubcoreMesh(core_axis_name="c", subcore_axis_name="s"),
    compiler_params=pltpu.CompilerParams(needs_layout_passes=False),
    out_type=jax.ShapeDtypeStruct((BATCH * VOCAB,), jnp.float32),
    scratch_types=[
        pltpu.VMEM((_L * SEQ,), jnp.int32),
        pltpu.VMEM((_L * VOCAB,), jnp.float32),
        pltpu.SemaphoreType.DMA,
    ],
)(_hist_body)


def _mm_body(counts_ref, w_ref, b_ref, out_ref):
    out_ref[...] = lax.dot_general(
        counts_ref[...], w_ref[...],
        dimension_numbers=(((1,), (1,)), ((), ())),
        preferred_element_type=jnp.float32,
    ) + b_ref[...]


_BM = 512


def _mm(counts, W, b2d):
    return pl.pallas_call(
        _mm_body,
        grid=(BATCH // _BM,),
        in_specs=[
            pl.BlockSpec((_BM, VOCAB), lambda i: (i, 0)),
            pl.BlockSpec((DMODEL, VOCAB), lambda i: (0, 0)),
            pl.BlockSpec((1, DMODEL), lambda i: (0, 0)),
        ],
        out_specs=pl.BlockSpec((_BM, DMODEL), lambda i: (i, 0)),
        out_shape=jax.ShapeDtypeStruct((BATCH, DMODEL), jnp.float32),
    )(counts, W, b2d)


def kernel(token_ids, W, b):
    tok_flat = token_ids.astype(jnp.int32).reshape(-1)
    counts = _hist(tok_flat).reshape(BATCH, VOCAB)
    out = _mm(counts, W, b.reshape(1, DMODEL))
    return out[:, None, :]


# speedup vs baseline: 23.3236x; 1.0083x over previous
import functools

import jax
import jax.numpy as jnp
from jax import lax
from jax.experimental import pallas as pl
from jax.experimental.pallas import tpu as pltpu
from jax.experimental.pallas import tpu_sc as plsc

BATCH = 4096
SEQ = 200
VOCAB = 512
DMODEL = 1024

_NC = 2
_NS = 16
_NW = _NC * _NS
_L = 16

_ROWS_PER_W = BATCH // _NW
_GROUPS = _ROWS_PER_W // _L


def _hist_body(tok_hbm, counts_hbm, tok_v, hist_v, sem):
    wid = lax.axis_index("s") * _NC + lax.axis_index("c")
    lane = lax.iota(jnp.int32, _L)
    zero16 = jnp.zeros((_L,), jnp.int32)
    ones = jnp.ones((_L,), jnp.float32)
    zeros = jnp.zeros((_L,), jnp.float32)

    for g in range(_GROUPS):
        base = (wid * _GROUPS + g) * _L

        pltpu.sync_copy(tok_hbm.at[pl.ds(base, _L), :], tok_v)

        def _zero(i, _):
            hist_v[i // (VOCAB // _L), pl.ds((i % (VOCAB // _L)) * _L, _L)] = zeros
            return 0
        lax.fori_loop(0, (_L * VOCAB) // _L, _zero, 0, unroll=8)

        def _step(s, _):
            tok = plsc.load_gather(tok_v, [lane, zero16 + s])
            plsc.addupdate_scatter(hist_v, [lane, tok], ones)
            return 0
        lax.fori_loop(0, SEQ, _step, 0, unroll=8)

        pltpu.sync_copy(hist_v, counts_hbm.at[pl.ds(base, _L), :])


_hist = functools.partial(
    pl.kernel,
    mesh=plsc.VectorSubcoreMesh(core_axis_name="c", subcore_axis_name="s"),
    compiler_params=pltpu.CompilerParams(needs_layout_passes=False),
    out_type=jax.ShapeDtypeStruct((BATCH, VOCAB), jnp.float32),
    scratch_types=[
        pltpu.VMEM((_L, SEQ), jnp.int32),
        pltpu.VMEM((_L, VOCAB), jnp.float32),
        pltpu.SemaphoreType.DMA,
    ],
)(_hist_body)


def _mm_body(counts_ref, w_ref, b_ref, out_ref):
    out_ref[...] = lax.dot_general(
        counts_ref[...], w_ref[...],
        dimension_numbers=(((1,), (1,)), ((), ())),
        preferred_element_type=jnp.float32,
    ) + b_ref[...]


_BM = 512


def _mm(counts, W, b2d):
    return pl.pallas_call(
        _mm_body,
        grid=(BATCH // _BM,),
        in_specs=[
            pl.BlockSpec((_BM, VOCAB), lambda i: (i, 0)),
            pl.BlockSpec((DMODEL, VOCAB), lambda i: (0, 0)),
            pl.BlockSpec((1, DMODEL), lambda i: (0, 0)),
        ],
        out_specs=pl.BlockSpec((_BM, DMODEL), lambda i: (i, 0)),
        out_shape=jax.ShapeDtypeStruct((BATCH, DMODEL), jnp.float32),
    )(counts, W, b2d)


def kernel(token_ids, W, b):
    tok = token_ids.astype(jnp.int32)
    counts = _hist(tok)
    out = _mm(counts, W, b.reshape(1, DMODEL))
    return out[:, None, :]


# speedup vs baseline: 39.0089x; 1.6725x over previous
import functools

import jax
import jax.numpy as jnp
from jax import lax
from jax.experimental import pallas as pl
from jax.experimental.pallas import tpu as pltpu
from jax.experimental.pallas import tpu_sc as plsc

BATCH = 4096
SEQ = 200
VOCAB = 512
DMODEL = 1024

_NC = 2
_NS = 16
_NW = _NC * _NS
_L = 16

_ROWS_PER_W = BATCH // _NW
_GROUPS = _ROWS_PER_W // _L


def _hist_body(tokT_hbm, counts_hbm, tok_v, hist_v, sem):
    wid = lax.axis_index("s") * _NC + lax.axis_index("c")
    lane = lax.iota(jnp.int32, _L)
    ones = jnp.ones((_L,), jnp.float32)
    zeros = jnp.zeros((_L,), jnp.float32)
    base = wid * _ROWS_PER_W

    pltpu.sync_copy(tokT_hbm.at[:, pl.ds(base, _ROWS_PER_W)], tok_v)

    def _zero(i, _):
        hist_v[i // (VOCAB // _L), pl.ds((i % (VOCAB // _L)) * _L, _L)] = zeros
        return 0
    lax.fori_loop(0, (_ROWS_PER_W * VOCAB) // _L, _zero, 0, unroll=16)

    for j in range(_ROWS_PER_W // _L):
        rows_j = j * _L + lane

        def _step(s, _):
            tok = tok_v[s, pl.ds(j * _L, _L)]
            plsc.addupdate_scatter(hist_v, [rows_j, tok], ones)
            return 0
        lax.fori_loop(0, SEQ, _step, 0, unroll=8)

    pltpu.sync_copy(hist_v, counts_hbm.at[pl.ds(base, _ROWS_PER_W), :])


_hist = functools.partial(
    pl.kernel,
    mesh=plsc.VectorSubcoreMesh(core_axis_name="c", subcore_axis_name="s"),
    compiler_params=pltpu.CompilerParams(needs_layout_passes=False),
    out_type=jax.ShapeDtypeStruct((BATCH, VOCAB), jnp.float32),
    scratch_types=[
        pltpu.VMEM((SEQ, _ROWS_PER_W), jnp.int32),
        pltpu.VMEM((_ROWS_PER_W, VOCAB), jnp.float32),
        pltpu.SemaphoreType.DMA,
    ],
)(_hist_body)


def _mm_body(counts_ref, w_ref, b_ref, out_ref):
    acc = lax.dot_general(
        counts_ref[...], w_ref[...],
        dimension_numbers=(((1,), (1,)), ((), ())),
        preferred_element_type=jnp.float32,
    ) + b_ref[...]
    out_ref[...] = acc[:, None, :]


_BM = 512


def _mm(counts, W, b2d):
    return pl.pallas_call(
        _mm_body,
        grid=(BATCH // _BM,),
        in_specs=[
            pl.BlockSpec((_BM, VOCAB), lambda i: (i, 0)),
            pl.BlockSpec((DMODEL, VOCAB), lambda i: (0, 0)),
            pl.BlockSpec((1, DMODEL), lambda i: (0, 0)),
        ],
        out_specs=pl.BlockSpec((_BM, 1, DMODEL), lambda i: (i, 0, 0)),
        out_shape=jax.ShapeDtypeStruct((BATCH, 1, DMODEL), jnp.float32),
    )(counts, W, b2d)


def kernel(token_ids, W, b):
    tokT = token_ids.astype(jnp.int32).T
    counts = _hist(tokT)
    return _mm(counts, W, b.reshape(1, DMODEL))
